# baseline (speedup 1.0000x reference)
import jax
import jax.numpy as jnp
from jax import lax
from jax.experimental import pallas as pl
from jax.experimental.pallas import tpu as pltpu
from jax.experimental.pallas import tpu_sc as plsc

NN = 10000
EE = 160000
DD = 256
HH = 4
CH = 256
FF = HH * CH
GG = 64
LRS = 0.2

NCORE = 2
NSUB = 16
NPASS = 20
RROWS = 512
HALF = RROWS // 2
NPAD = NPASS * RROWS
GROW = HALF
AROWS = HALF + 8
EPT = EE // NSUB
BATCH = 64
ESEG = 2000
NSEG = EPT // ESEG
SCHUNK = ESEG // 16
MCAP = 2192
MDUMP = 2144
BR = 1000


def _attn_outputs(h, amap_ref, as_ref, ad_ref, el_ref):
    sa = jnp.dot(h, amap_ref[...], preferred_element_type=jnp.float32)
    as_ref[...] = sa[:, 0:4]
    ad_ref[...] = sa[:, 4:8]
    e = sa[:, 0:4] + sa[:, 4:8]
    el_ref[...] = jnp.exp(jnp.maximum(e, LRS * e))


def _tc_a1_body(x_ref, w_ref, amap_ref, h_ref, as_ref, ad_ref, el_ref):
    h = jnp.dot(x_ref[...], w_ref[...], preferred_element_type=jnp.float32)
    h_ref[...] = h
    _attn_outputs(h, amap_ref, as_ref, ad_ref, el_ref)


def _tc_a1(x, W, amap):
    return pl.pallas_call(
        _tc_a1_body,
        grid=(NN // BR,),
        in_specs=[
            pl.BlockSpec((BR, DD), lambda i: (i, 0)),
            pl.BlockSpec((DD, FF), lambda i: (0, 0)),
            pl.BlockSpec((FF, 8), lambda i: (0, 0)),
        ],
        out_specs=[
            pl.BlockSpec((BR, FF), lambda i: (i, 0)),
            pl.BlockSpec((BR, 4), lambda i: (i, 0)),
            pl.BlockSpec((BR, 4), lambda i: (i, 0)),
            pl.BlockSpec((BR, 4), lambda i: (i, 0)),
        ],
        out_shape=[
            jax.ShapeDtypeStruct((NN, FF), jnp.float32),
            jax.ShapeDtypeStruct((NPAD, 4), jnp.float32),
            jax.ShapeDtypeStruct((NPAD, 4), jnp.float32),
            jax.ShapeDtypeStruct((NN, 4), jnp.float32),
        ],
    )(x, W, amap)


def _combine(acc3, den8, el, h, b):
    sel = (lax.broadcasted_iota(jnp.int32, (8, 4), 0) ==
           2 * lax.broadcasted_iota(jnp.int32, (8, 4), 1))
    den4 = jnp.dot(den8, sel.astype(jnp.float32),
                   preferred_element_type=jnp.float32)
    ones128 = jnp.ones((1, 128), jnp.float32)
    outs = []
    for blk in range(8):
        hh = blk // 2
        el_h = el[:, hh:hh + 1]
        num = acc3[blk] + jnp.dot(el_h, ones128,
                                  preferred_element_type=jnp.float32) \
            * h[:, blk * 128:(blk + 1) * 128]
        dsum = den4[:, hh:hh + 1] + el_h + 1e-16
        outs.append(num / jnp.dot(dsum, ones128,
                                  preferred_element_type=jnp.float32))
    out = jnp.concatenate(outs, axis=1)
    return jnp.maximum(out + b, 0.0)


def _tc_a2_body(acc_ref, den_ref, el_ref, h_ref, b_ref, w_ref,
                amap_ref, h2_ref, as_ref, ad_ref, el2_ref):
    out1 = _combine(acc_ref[...], den_ref[...], el_ref[...], h_ref[...],
                    b_ref[...])
    h2 = jnp.dot(out1, w_ref[...], preferred_element_type=jnp.float32)
    h2_ref[...] = h2
    _attn_outputs(h2, amap_ref, as_ref, ad_ref, el2_ref)


def _tc_a2(acc3, den8, el, h, b, W, amap):
    return pl.pallas_call(
        _tc_a2_body,
        grid=(NN // BR,),
        in_specs=[
            pl.BlockSpec((8, BR, 128), lambda i: (0, i, 0)),
            pl.BlockSpec((BR, 8), lambda i: (i, 0)),
            pl.BlockSpec((BR, 4), lambda i: (i, 0)),
            pl.BlockSpec((BR, FF), lambda i: (i, 0)),
            pl.BlockSpec((1, FF), lambda i: (0, 0)),
            pl.BlockSpec((FF, FF), lambda i: (0, 0)),
            pl.BlockSpec((FF, 8), lambda i: (0, 0)),
        ],
        out_specs=[
            pl.BlockSpec((BR, FF), lambda i: (i, 0)),
            pl.BlockSpec((BR, 4), lambda i: (i, 0)),
            pl.BlockSpec((BR, 4), lambda i: (i, 0)),
            pl.BlockSpec((BR, 4), lambda i: (i, 0)),
        ],
        out_shape=[
            jax.ShapeDtypeStruct((NN, FF), jnp.float32),
            jax.ShapeDtypeStruct((NPAD, 4), jnp.float32),
            jax.ShapeDtypeStruct((NPAD, 4), jnp.float32),
            jax.ShapeDtypeStruct((NN, 4), jnp.float32),
        ],
    )(acc3, den8, el, h, b, W, amap)


def _tc_d_body(acc_ref, den_ref, el_ref, h_ref, b_ref, batch_ref,
               fc1w_ref, fc1b_ref, fc2w_ref, fc2b_ref, out_ref,
               sums_ref, cnt_ref):
    j = pl.program_id(0)
    out2 = _combine(acc_ref[...], den_ref[...], el_ref[...], h_ref[...],
                    b_ref[...])
    gids = lax.broadcasted_iota(jnp.int32, (GG, BR), 0)
    m = (gids == batch_ref[0]).astype(jnp.float32)
    psum = jnp.dot(m, out2, preferred_element_type=jnp.float32)
    pcnt = jnp.dot(m, jnp.ones((BR, 128), jnp.float32),
                   preferred_element_type=jnp.float32)

    @pl.when(j == 0)
    def _():
        sums_ref[...] = psum
        cnt_ref[...] = pcnt

    @pl.when(j > 0)
    def _():
        sums_ref[...] = sums_ref[...] + psum
        cnt_ref[...] = cnt_ref[...] + pcnt

    @pl.when(j == NN // BR - 1)
    def _():
        cnt = jnp.maximum(cnt_ref[:, 0:1], 1.0)
        cntb = jnp.dot(cnt, jnp.ones((1, FF), jnp.float32),
                       preferred_element_type=jnp.float32)
        pooled = sums_ref[...] / cntb
        z = jnp.dot(pooled, fc1w_ref[...], preferred_element_type=jnp.float32)
        z = jnp.maximum(z + fc1b_ref[...], 0.0)
        logits = jnp.dot(z, fc2w_ref[...], preferred_element_type=jnp.float32)
        logits = logits + fc2b_ref[...]
        mx = jnp.max(logits, axis=1, keepdims=True)
        s = logits - mx
        out_ref[...] = s - jnp.log(jnp.sum(jnp.exp(s), axis=1, keepdims=True))


def _tc_d(acc3, den8, el, h, b, batch3d, fc1_W, fc1_b, fc2_W, fc2_b):
    return pl.pallas_call(
        _tc_d_body,
        grid=(NN // BR,),
        in_specs=[
            pl.BlockSpec((8, BR, 128), lambda i: (0, i, 0)),
            pl.BlockSpec((BR, 8), lambda i: (i, 0)),
            pl.BlockSpec((BR, 4), lambda i: (i, 0)),
            pl.BlockSpec((BR, FF), lambda i: (i, 0)),
            pl.BlockSpec((1, FF), lambda i: (0, 0)),
            pl.BlockSpec((1, 1, BR), lambda i: (i, 0, 0)),
            pl.BlockSpec((FF, CH), lambda i: (0, 0)),
            pl.BlockSpec((1, CH), lambda i: (0, 0)),
            pl.BlockSpec((CH, 10), lambda i: (0, 0)),
            pl.BlockSpec((1, 10), lambda i: (0, 0)),
        ],
        out_specs=pl.BlockSpec((GG, 10), lambda i: (0, 0)),
        out_shape=jax.ShapeDtypeStruct((GG, 10), jnp.float32),
        scratch_shapes=[
            pltpu.VMEM((GG, FF), jnp.float32),
            pltpu.VMEM((GG, 128), jnp.float32),
        ],
    )(acc3, den8, el, h, b, batch3d, fc1_W, fc1_b, fc2_W, fc2_b)


def _sc_edge_body(src_hbm, dst_hbm, as_hbm, ad_hbm, h2d_hbm, acc_hbm, den_hbm,
                  esrc, edst, srcma, dstma, srcmb, dstmb, astab, adtab,
                  cntv, cbuf, srcs32, dsts32, srcs32b, dsts32b, eebuf,
                  rows, rowsb, acc_v, den_v,
                  wsrc_s, wdst_s, cnt_s, sem0, sem1):
    cid = lax.axis_index("c")
    sid = lax.axis_index("s")
    blk = lax.bitwise_and(sid, 7)
    par = lax.shift_right_logical(sid, 3)
    hh = lax.shift_right_logical(blk, 1)
    iota = lax.iota(jnp.int32, 16)
    zeros16 = jnp.zeros((16,), jnp.float32)
    zeros16i = jnp.zeros((16,), jnp.int32)
    grow16 = jnp.full((16,), GROW, jnp.int32)

    ebase = sid * EPT
    pltpu.sync_copy(as_hbm.at[pl.ds(0, NN * 4)], astab)

    def process_lists():
        pltpu.sync_copy(cnt_s, cbuf)

        def body_for(k, carry):
            ck = plsc.load_gather(
                cbuf, [jnp.full((16,), k * 16 + par, jnp.int32)])[0]
            lofs = (par * 16 + k) * MCAP
            ra = pltpu.async_copy(wsrc_s.at[pl.ds(lofs, MCAP)],
                                  esrc.at[pl.ds(0, MCAP)], sem0)
            rb = pltpu.async_copy(wdst_s.at[pl.ds(lofs, MCAP)],
                                  edst.at[pl.ds(0, MCAP)], sem1)
            ra.wait()
            rb.wait()

            def build(b, srcsX, dstsX):
                boff = b * BATCH
                for j in range(BATCH // 16):
                    sv = esrc[pl.ds(boff + j * 16, 16)]
                    dl = edst[pl.ds(boff + j * 16, 16)]
                    srcsX[pl.ds(j * 16, 16)] = sv * 8 + blk
                    dstsX[pl.ds(j * 16, 16)] = dl
                return None

            def compute(b, dstsX, rowsX):
                boff = b * BATCH
                for j in range(BATCH // 16):
                    sv = esrc[pl.ds(boff + j * 16, 16)]
                    dl = edst[pl.ds(boff + j * 16, 16)]
                    a = plsc.load_gather(astab, [sv * 4 + hh])
                    d = plsc.load_gather(
                        adtab, [(par * HALF + dl) * 4 + hh])
                    e = a + d
                    eebuf[pl.ds(j * 16, 16)] = jnp.exp(
                        jnp.maximum(e, LRS * e))

                def _edge(r, carry3):
                    sp = plsc.load_gather(eebuf,
                                          [jnp.full((16,), r, jnp.int32)])
                    dl = plsc.load_gather(dstsX,
                                          [jnp.full((16,), r, jnp.int32)])
                    rbase = dl * 128
                    rfull = jnp.full((16,), r, jnp.int32)
                    for c in range(8):
                        colv = c * 16 + iota
                        hv = plsc.load_gather(rowsX, [rfull, colv])
                        g = plsc.load_gather(acc_v, [rbase + colv])
                        plsc.store_scatter(acc_v, [rbase + colv],
                                           g + sp * hv)
                    dv = plsc.load_gather(den_v, [dl])
                    plsc.store_scatter(den_v, [dl], dv + sp)
                    return carry3
                lax.fori_loop(0, BATCH, _edge, 0)

            np2 = (ck + 2 * BATCH - 1) // (2 * BATCH)

            def _pair(pp, carry2):
                b0 = 2 * pp
                build(b0, srcs32, dsts32)
                g0 = pltpu.async_copy(h2d_hbm.at[srcs32], rows, sem0)
                build(b0 + 1, srcs32b, dsts32b)
                g1 = pltpu.async_copy(h2d_hbm.at[srcs32b], rowsb, sem1)
                g0.wait()
                compute(b0, dsts32, rows)
                g1.wait()
                compute(b0 + 1, dsts32b, rowsb)
                return carry2
            lax.fori_loop(0, np2, _pair, 0)
            return carry
        lax.fori_loop(0, 16, body_for, 0)

    def run_pass(p):
        lo = p * RROWS

        def _zacc(j, carry):
            acc_v[pl.ds(j * 16, 16)] = zeros16
            return carry
        lax.fori_loop(0, AROWS * 8, _zacc, 0)
        for j in range(AROWS // 16 + 1):
            den_v[pl.ds(j * 16, 16)] = zeros16

        pltpu.sync_copy(ad_hbm.at[pl.ds(lo * 4, RROWS * 4)],
                        adtab.at[pl.ds(0, RROWS * 4)])

        def scan_chunk(c, cnts):
            cnta, cntb = cnts
            s16 = esrc[pl.ds(c * 16, 16)]
            d16 = edst[pl.ds(c * 16, 16)]
            dl = d16 - lo
            for half in range(2):
                msk = (dl >= half * HALF) & (dl < (half + 1) * HALF)
                cnt = cnta if half == 0 else cntb
                pref = jnp.where(msk, 1, 0)
                for k in (1, 2, 4, 8):
                    shifted = pref.at[jnp.maximum(iota - k, 0)].get(
                        mode="promise_in_bounds")
                    pref = pref + jnp.where(iota >= k, shifted, 0)
                pos = jnp.where(msk, cnt + pref - 1, MDUMP)
                if half == 0:
                    plsc.store_scatter(srcma, [pos], s16)
                    plsc.store_scatter(dstma, [pos], dl)
                    cnta = cnt + pref[15]
                else:
                    plsc.store_scatter(srcmb, [pos], s16)
                    plsc.store_scatter(dstmb, [pos], dl - HALF)
                    cntb = cnt + pref[15]
            return (cnta, cntb)

        def _segment(seg, carry):
            pltpu.sync_copy(
                src_hbm.at[pl.ds(ebase + seg * ESEG, ESEG)],
                esrc.at[pl.ds(0, ESEG)])
            pltpu.sync_copy(
                dst_hbm.at[pl.ds(ebase + seg * ESEG, ESEG)],
                edst.at[pl.ds(0, ESEG)])
            cnta, cntb = lax.fori_loop(
                0, SCHUNK, scan_chunk, (jnp.int32(0), jnp.int32(0)))
            for j in range(2 * BATCH // 16):
                srcma[pl.ds(cnta + j * 16, 16)] = zeros16i
                dstma[pl.ds(cnta + j * 16, 16)] = grow16
                srcmb[pl.ds(cntb + j * 16, 16)] = zeros16i
                dstmb[pl.ds(cntb + j * 16, 16)] = grow16
            pltpu.sync_copy(srcma, wsrc_s.at[pl.ds(sid * MCAP, MCAP)])
            pltpu.sync_copy(dstma, wdst_s.at[pl.ds(sid * MCAP, MCAP)])
            pltpu.sync_copy(srcmb,
                            wsrc_s.at[pl.ds((16 + sid) * MCAP, MCAP)])
            pltpu.sync_copy(dstmb,
                            wdst_s.at[pl.ds((16 + sid) * MCAP, MCAP)])
            cntv[...] = jnp.where(iota == 0, cnta,
                                  jnp.where(iota == 1, cntb, 0))
            pltpu.sync_copy(cntv, cnt_s.at[pl.ds(sid * 16, 16)])
            plsc.subcore_barrier()
            process_lists()
            plsc.subcore_barrier()
            return carry
        lax.fori_loop(0, NSEG, _segment, 0)

        rbase = blk * (NPAD * 128) + (lo + par * HALF) * 128
        pltpu.sync_copy(acc_v.at[pl.ds(0, HALF * 128)],
                        acc_hbm.at[pl.ds(rbase, HALF * 128)])
        pltpu.sync_copy(den_v.at[pl.ds(0, HALF)],
                        den_hbm.at[pl.ds(blk * NPAD + lo + par * HALF,
                                         HALF)])

    def _pass(p3, carry):
        run_pass(p3 * NCORE + cid)
        return carry
    lax.fori_loop(0, NPASS // NCORE, _pass, 0)


def _sc_edge(src, dst, asf, adf, h2d):
    mesh = plsc.VectorSubcoreMesh(core_axis_name="c", subcore_axis_name="s")
    return pl.kernel(
        _sc_edge_body,
        out_type=[
            jax.ShapeDtypeStruct((8 * NPAD * 128,), jnp.float32),
            jax.ShapeDtypeStruct((8 * NPAD,), jnp.float32),
        ],
        mesh=mesh,
        compiler_params=pltpu.CompilerParams(needs_layout_passes=False),
        scratch_types=[
            pltpu.VMEM((MCAP + 16,), jnp.int32),
            pltpu.VMEM((MCAP + 16,), jnp.int32),
            pltpu.VMEM((MCAP,), jnp.int32),
            pltpu.VMEM((MCAP,), jnp.int32),
            pltpu.VMEM((MCAP,), jnp.int32),
            pltpu.VMEM((MCAP,), jnp.int32),
            pltpu.VMEM((NN * 4,), jnp.float32),
            pltpu.VMEM((RROWS * 4 + 48,), jnp.float32),
            pltpu.VMEM((16,), jnp.int32),
            pltpu.VMEM((256,), jnp.int32),
            pltpu.VMEM((BATCH,), jnp.int32),
            pltpu.VMEM((BATCH,), jnp.int32),
            pltpu.VMEM((BATCH,), jnp.int32),
            pltpu.VMEM((BATCH,), jnp.int32),
            pltpu.VMEM((BATCH,), jnp.float32),
            pltpu.VMEM((BATCH, 128), jnp.float32),
            pltpu.VMEM((BATCH, 128), jnp.float32),
            pltpu.VMEM((AROWS * 128,), jnp.float32),
            pltpu.VMEM((AROWS + 8,), jnp.float32),
            pltpu.VMEM_SHARED((32 * MCAP,), jnp.int32),
            pltpu.VMEM_SHARED((32 * MCAP,), jnp.int32),
            pltpu.VMEM_SHARED((256,), jnp.int32),
            pltpu.SemaphoreType.DMA,
            pltpu.SemaphoreType.DMA,
        ],
    )(src, dst, asf, adf, h2d)


def kernel(x, edge_index, batch, W1, a_src1, a_dst1, b1, W2, a_src2, a_dst2,
           b2, fc1_W, fc1_b, fc2_W, fc2_b):
    src = edge_index[0]
    dst = edge_index[1]

    def _amap(a_src, a_dst):
        eye = jnp.eye(HH, dtype=jnp.float32)
        blk = jnp.repeat(eye, CH, axis=0)
        asrc_col = a_src.reshape(FF, 1) * blk
        adst_col = a_dst.reshape(FF, 1) * blk
        return jnp.concatenate([asrc_col, adst_col], axis=1)

    amap1 = _amap(a_src1, a_dst1)
    amap2 = _amap(a_src2, a_dst2)

    h1, as1, ad1, el1 = _tc_a1(x, W1, amap1)
    acc1, den1 = _sc_edge(src, dst, as1.reshape(NPAD * 4),
                          ad1.reshape(NPAD * 4), h1.reshape(NN * 8, 128))
    h2, as2, ad2, el2 = _tc_a2(acc1.reshape(8, NPAD, 128),
                               den1.reshape(8, NPAD).T, el1, h1,
                               b1.reshape(1, FF), W2, amap2)
    acc2, den2 = _sc_edge(src, dst, as2.reshape(NPAD * 4),
                          ad2.reshape(NPAD * 4), h2.reshape(NN * 8, 128))
    out = _tc_d(acc2.reshape(8, NPAD, 128), den2.reshape(8, NPAD).T, el2,
                h2,
                b2.reshape(1, FF), batch.reshape(NN // BR, 1, BR),
                fc1_W, fc1_b.reshape(1, CH), fc2_W, fc2_b.reshape(1, 10))
    return out


# speedup vs baseline: 2.1757x; 2.1757x over previous
import jax
import jax.numpy as jnp
from jax import lax
from jax.experimental import pallas as pl
from jax.experimental.pallas import tpu as pltpu
from jax.experimental.pallas import tpu_sc as plsc

NN = 10000
EE = 160000
DD = 256
HH = 4
CH = 256
FF = HH * CH
GG = 64
LRS = 0.2

NCORE = 2
NSUB = 16
NPASS = 20
RROWS = 512
HALF = RROWS // 2
NPAD = NPASS * RROWS
GROW = HALF
AROWS = HALF + 8
EPT = EE // NSUB
BATCH = 64
ESEG = 2000
NSEG = EPT // ESEG
SCHUNK = ESEG // 16
MCAP = 2192
MDUMP = 2144
BR = 1000


def _attn_outputs(h, amap_ref, as_ref, ad_ref, el_ref):
    sa = jnp.dot(h, amap_ref[...], preferred_element_type=jnp.float32)
    as_ref[...] = sa[:, 0:4]
    ad_ref[...] = sa[:, 4:8]
    e = sa[:, 0:4] + sa[:, 4:8]
    el_ref[...] = jnp.exp(jnp.maximum(e, LRS * e))


def _tc_a1_body(x_ref, w_ref, amap_ref, h_ref, as_ref, ad_ref, el_ref):
    h = jnp.dot(x_ref[...], w_ref[...], preferred_element_type=jnp.float32)
    h_ref[...] = h
    _attn_outputs(h, amap_ref, as_ref, ad_ref, el_ref)


def _tc_a1(x, W, amap):
    return pl.pallas_call(
        _tc_a1_body,
        grid=(NN // BR,),
        in_specs=[
            pl.BlockSpec((BR, DD), lambda i: (i, 0)),
            pl.BlockSpec((DD, FF), lambda i: (0, 0)),
            pl.BlockSpec((FF, 8), lambda i: (0, 0)),
        ],
        out_specs=[
            pl.BlockSpec((BR, FF), lambda i: (i, 0)),
            pl.BlockSpec((BR, 4), lambda i: (i, 0)),
            pl.BlockSpec((BR, 4), lambda i: (i, 0)),
            pl.BlockSpec((BR, 4), lambda i: (i, 0)),
        ],
        out_shape=[
            jax.ShapeDtypeStruct((NN, FF), jnp.float32),
            jax.ShapeDtypeStruct((NPAD, 4), jnp.float32),
            jax.ShapeDtypeStruct((NPAD, 4), jnp.float32),
            jax.ShapeDtypeStruct((NN, 4), jnp.float32),
        ],
    )(x, W, amap)


def _combine(acc3, den8, el, h, b):
    sel = (lax.broadcasted_iota(jnp.int32, (8, 4), 0) ==
           2 * lax.broadcasted_iota(jnp.int32, (8, 4), 1))
    den4 = jnp.dot(den8, sel.astype(jnp.float32),
                   preferred_element_type=jnp.float32)
    ones128 = jnp.ones((1, 128), jnp.float32)
    outs = []
    for blk in range(8):
        hh = blk // 2
        el_h = el[:, hh:hh + 1]
        num = acc3[blk] + jnp.dot(el_h, ones128,
                                  preferred_element_type=jnp.float32) \
            * h[:, blk * 128:(blk + 1) * 128]
        dsum = den4[:, hh:hh + 1] + el_h + 1e-16
        outs.append(num / jnp.dot(dsum, ones128,
                                  preferred_element_type=jnp.float32))
    out = jnp.concatenate(outs, axis=1)
    return jnp.maximum(out + b, 0.0)


def _tc_a2_body(acc_ref, den_ref, el_ref, h_ref, b_ref, w_ref,
                amap_ref, h2_ref, as_ref, ad_ref, el2_ref):
    out1 = _combine(acc_ref[...], den_ref[...], el_ref[...], h_ref[...],
                    b_ref[...])
    h2 = jnp.dot(out1, w_ref[...], preferred_element_type=jnp.float32)
    h2_ref[...] = h2
    _attn_outputs(h2, amap_ref, as_ref, ad_ref, el2_ref)


def _tc_a2(acc3, den8, el, h, b, W, amap):
    return pl.pallas_call(
        _tc_a2_body,
        grid=(NN // BR,),
        in_specs=[
            pl.BlockSpec((8, BR, 128), lambda i: (0, i, 0)),
            pl.BlockSpec((BR, 8), lambda i: (i, 0)),
            pl.BlockSpec((BR, 4), lambda i: (i, 0)),
            pl.BlockSpec((BR, FF), lambda i: (i, 0)),
            pl.BlockSpec((1, FF), lambda i: (0, 0)),
            pl.BlockSpec((FF, FF), lambda i: (0, 0)),
            pl.BlockSpec((FF, 8), lambda i: (0, 0)),
        ],
        out_specs=[
            pl.BlockSpec((BR, FF), lambda i: (i, 0)),
            pl.BlockSpec((BR, 4), lambda i: (i, 0)),
            pl.BlockSpec((BR, 4), lambda i: (i, 0)),
            pl.BlockSpec((BR, 4), lambda i: (i, 0)),
        ],
        out_shape=[
            jax.ShapeDtypeStruct((NN, FF), jnp.float32),
            jax.ShapeDtypeStruct((NPAD, 4), jnp.float32),
            jax.ShapeDtypeStruct((NPAD, 4), jnp.float32),
            jax.ShapeDtypeStruct((NN, 4), jnp.float32),
        ],
    )(acc3, den8, el, h, b, W, amap)


def _tc_d_body(acc_ref, den_ref, el_ref, h_ref, b_ref, batch_ref,
               fc1w_ref, fc1b_ref, fc2w_ref, fc2b_ref, out_ref,
               sums_ref, cnt_ref):
    j = pl.program_id(0)
    out2 = _combine(acc_ref[...], den_ref[...], el_ref[...], h_ref[...],
                    b_ref[...])
    gids = lax.broadcasted_iota(jnp.int32, (GG, BR), 0)
    m = (gids == batch_ref[0]).astype(jnp.float32)
    psum = jnp.dot(m, out2, preferred_element_type=jnp.float32)
    pcnt = jnp.dot(m, jnp.ones((BR, 128), jnp.float32),
                   preferred_element_type=jnp.float32)

    @pl.when(j == 0)
    def _():
        sums_ref[...] = psum
        cnt_ref[...] = pcnt

    @pl.when(j > 0)
    def _():
        sums_ref[...] = sums_ref[...] + psum
        cnt_ref[...] = cnt_ref[...] + pcnt

    @pl.when(j == NN // BR - 1)
    def _():
        cnt = jnp.maximum(cnt_ref[:, 0:1], 1.0)
        cntb = jnp.dot(cnt, jnp.ones((1, FF), jnp.float32),
                       preferred_element_type=jnp.float32)
        pooled = sums_ref[...] / cntb
        z = jnp.dot(pooled, fc1w_ref[...], preferred_element_type=jnp.float32)
        z = jnp.maximum(z + fc1b_ref[...], 0.0)
        logits = jnp.dot(z, fc2w_ref[...], preferred_element_type=jnp.float32)
        logits = logits + fc2b_ref[...]
        mx = jnp.max(logits, axis=1, keepdims=True)
        s = logits - mx
        out_ref[...] = s - jnp.log(jnp.sum(jnp.exp(s), axis=1, keepdims=True))


def _tc_d(acc3, den8, el, h, b, batch3d, fc1_W, fc1_b, fc2_W, fc2_b):
    return pl.pallas_call(
        _tc_d_body,
        grid=(NN // BR,),
        in_specs=[
            pl.BlockSpec((8, BR, 128), lambda i: (0, i, 0)),
            pl.BlockSpec((BR, 8), lambda i: (i, 0)),
            pl.BlockSpec((BR, 4), lambda i: (i, 0)),
            pl.BlockSpec((BR, FF), lambda i: (i, 0)),
            pl.BlockSpec((1, FF), lambda i: (0, 0)),
            pl.BlockSpec((1, 1, BR), lambda i: (i, 0, 0)),
            pl.BlockSpec((FF, CH), lambda i: (0, 0)),
            pl.BlockSpec((1, CH), lambda i: (0, 0)),
            pl.BlockSpec((CH, 10), lambda i: (0, 0)),
            pl.BlockSpec((1, 10), lambda i: (0, 0)),
        ],
        out_specs=pl.BlockSpec((GG, 10), lambda i: (0, 0)),
        out_shape=jax.ShapeDtypeStruct((GG, 10), jnp.float32),
        scratch_shapes=[
            pltpu.VMEM((GG, FF), jnp.float32),
            pltpu.VMEM((GG, 128), jnp.float32),
        ],
    )(acc3, den8, el, h, b, batch3d, fc1_W, fc1_b, fc2_W, fc2_b)


def _sc_edge_body(src_hbm, dst_hbm, as_hbm, ad_hbm, h2d_hbm, acc_hbm, den_hbm,
                  esrc, edst, srcma, dstma, srcmb, dstmb, astab, adtab,
                  cntv, cbuf, srcs32, dsts32, srcs32b, dsts32b, eebuf,
                  rows, rowsb, acc_v, den_v,
                  wsrc_s, wdst_s, cnt_s, sem0, sem1):
    cid = lax.axis_index("c")
    sid = lax.axis_index("s")
    blk = lax.bitwise_and(sid, 7)
    par = lax.shift_right_logical(sid, 3)
    hh = lax.shift_right_logical(blk, 1)
    iota = lax.iota(jnp.int32, 16)
    zeros16 = jnp.zeros((16,), jnp.float32)
    zeros16i = jnp.zeros((16,), jnp.int32)
    grow16 = jnp.full((16,), GROW, jnp.int32)

    ebase = sid * EPT
    pltpu.sync_copy(as_hbm.at[pl.ds(0, NN * 4)], astab)

    def process_lists():
        pltpu.sync_copy(cnt_s, cbuf)

        def body_for(k, carry):
            ck = plsc.load_gather(
                cbuf, [jnp.full((16,), k * 16 + par, jnp.int32)])[0]
            lofs = (par * 16 + k) * MCAP
            ra = pltpu.async_copy(wsrc_s.at[pl.ds(lofs, MCAP)],
                                  esrc.at[pl.ds(0, MCAP)], sem0)
            rb = pltpu.async_copy(wdst_s.at[pl.ds(lofs, MCAP)],
                                  edst.at[pl.ds(0, MCAP)], sem1)
            ra.wait()
            rb.wait()

            def build(b, srcsX, dstsX):
                boff = b * BATCH
                for j in range(BATCH // 16):
                    sv = esrc[pl.ds(boff + j * 16, 16)]
                    dl = edst[pl.ds(boff + j * 16, 16)]
                    srcsX[pl.ds(j * 16, 16)] = sv * 8 + blk
                    dstsX[pl.ds(j * 16, 16)] = dl
                return None

            def compute(b, dstsX, rowsX):
                boff = b * BATCH
                for j in range(BATCH // 16):
                    sv = esrc[pl.ds(boff + j * 16, 16)]
                    dl = edst[pl.ds(boff + j * 16, 16)]
                    a = plsc.load_gather(astab, [sv * 4 + hh])
                    d = plsc.load_gather(
                        adtab, [(par * HALF + dl) * 4 + hh])
                    e = a + d
                    eebuf[pl.ds(j * 16, 16)] = jnp.exp(
                        jnp.maximum(e, LRS * e))

                def _edge(r, carry3):
                    sp = plsc.load_gather(eebuf,
                                          [jnp.full((16,), r, jnp.int32)])
                    dl = plsc.load_gather(dstsX,
                                          [jnp.full((16,), r, jnp.int32)])
                    rbase = dl * 128
                    rfull = jnp.full((16,), r, jnp.int32)
                    for c in range(8):
                        colv = c * 16 + iota
                        hv = plsc.load_gather(rowsX, [rfull, colv])
                        g = plsc.load_gather(acc_v, [rbase + colv])
                        plsc.store_scatter(acc_v, [rbase + colv],
                                           g + sp * hv)
                    dv = plsc.load_gather(den_v, [dl])
                    plsc.store_scatter(den_v, [dl], dv + sp)
                    return carry3
                lax.fori_loop(0, BATCH, _edge, 0)

            nb = (ck + BATCH - 1) // BATCH

            def _sub(b, carry2):
                build(b, srcs32, dsts32)
                pltpu.async_copy(h2d_hbm.at[srcs32], rows, sem0).wait()
                compute(b, dsts32, rows)
                return carry2
            lax.fori_loop(0, nb, _sub, 0)
            return carry
        lax.fori_loop(0, 16, body_for, 0)

    def run_pass(p):
        lo = p * RROWS

        def _zacc(j, carry):
            acc_v[pl.ds(j * 16, 16)] = zeros16
            return carry
        lax.fori_loop(0, AROWS * 8, _zacc, 0)
        for j in range(AROWS // 16 + 1):
            den_v[pl.ds(j * 16, 16)] = zeros16

        pltpu.sync_copy(ad_hbm.at[pl.ds(lo * 4, RROWS * 4)],
                        adtab.at[pl.ds(0, RROWS * 4)])

        def scan_chunk(c, cnts):
            cnta, cntb = cnts
            s16 = esrc[pl.ds(c * 16, 16)]
            d16 = edst[pl.ds(c * 16, 16)]
            dl = d16 - lo
            for half in range(2):
                msk = (dl >= half * HALF) & (dl < (half + 1) * HALF)
                cnt = cnta if half == 0 else cntb
                pref = jnp.where(msk, 1, 0)
                for k in (1, 2, 4, 8):
                    shifted = pref.at[jnp.maximum(iota - k, 0)].get(
                        mode="promise_in_bounds")
                    pref = pref + jnp.where(iota >= k, shifted, 0)
                pos = jnp.where(msk, cnt + pref - 1, MDUMP)
                if half == 0:
                    plsc.store_scatter(srcma, [pos], s16)
                    plsc.store_scatter(dstma, [pos], dl)
                    cnta = cnt + pref[15]
                else:
                    plsc.store_scatter(srcmb, [pos], s16)
                    plsc.store_scatter(dstmb, [pos], dl - HALF)
                    cntb = cnt + pref[15]
            return (cnta, cntb)

        def _segment(seg, carry):
            pltpu.sync_copy(
                src_hbm.at[pl.ds(ebase + seg * ESEG, ESEG)],
                esrc.at[pl.ds(0, ESEG)])
            pltpu.sync_copy(
                dst_hbm.at[pl.ds(ebase + seg * ESEG, ESEG)],
                edst.at[pl.ds(0, ESEG)])
            cnta, cntb = lax.fori_loop(
                0, SCHUNK, scan_chunk, (jnp.int32(0), jnp.int32(0)))
            for j in range(2 * BATCH // 16):
                srcma[pl.ds(cnta + j * 16, 16)] = zeros16i
                dstma[pl.ds(cnta + j * 16, 16)] = grow16
                srcmb[pl.ds(cntb + j * 16, 16)] = zeros16i
                dstmb[pl.ds(cntb + j * 16, 16)] = grow16
            pltpu.sync_copy(srcma, wsrc_s.at[pl.ds(sid * MCAP, MCAP)])
            pltpu.sync_copy(dstma, wdst_s.at[pl.ds(sid * MCAP, MCAP)])
            pltpu.sync_copy(srcmb,
                            wsrc_s.at[pl.ds((16 + sid) * MCAP, MCAP)])
            pltpu.sync_copy(dstmb,
                            wdst_s.at[pl.ds((16 + sid) * MCAP, MCAP)])
            cntv[...] = jnp.where(iota == 0, cnta,
                                  jnp.where(iota == 1, cntb, 0))
            pltpu.sync_copy(cntv, cnt_s.at[pl.ds(sid * 16, 16)])
            plsc.subcore_barrier()
            process_lists()
            plsc.subcore_barrier()
            return carry
        lax.fori_loop(0, NSEG, _segment, 0)

        rbase = blk * (NPAD * 128) + (lo + par * HALF) * 128
        pltpu.sync_copy(acc_v.at[pl.ds(0, HALF * 128)],
                        acc_hbm.at[pl.ds(rbase, HALF * 128)])
        pltpu.sync_copy(den_v.at[pl.ds(0, HALF)],
                        den_hbm.at[pl.ds(blk * NPAD + lo + par * HALF,
                                         HALF)])

    def _pass(p3, carry):
        run_pass(p3 * NCORE + cid)
        return carry
    lax.fori_loop(0, NPASS // NCORE, _pass, 0)


def _sc_edge(src, dst, asf, adf, h2d):
    mesh = plsc.VectorSubcoreMesh(core_axis_name="c", subcore_axis_name="s")
    return pl.kernel(
        _sc_edge_body,
        out_type=[
            jax.ShapeDtypeStruct((8 * NPAD * 128,), jnp.float32),
            jax.ShapeDtypeStruct((8 * NPAD,), jnp.float32),
        ],
        mesh=mesh,
        compiler_params=pltpu.CompilerParams(needs_layout_passes=False),
        scratch_types=[
            pltpu.VMEM((MCAP + 16,), jnp.int32),
            pltpu.VMEM((MCAP + 16,), jnp.int32),
            pltpu.VMEM((MCAP,), jnp.int32),
            pltpu.VMEM((MCAP,), jnp.int32),
            pltpu.VMEM((MCAP,), jnp.int32),
            pltpu.VMEM((MCAP,), jnp.int32),
            pltpu.VMEM((NN * 4,), jnp.float32),
            pltpu.VMEM((RROWS * 4 + 48,), jnp.float32),
            pltpu.VMEM((16,), jnp.int32),
            pltpu.VMEM((256,), jnp.int32),
            pltpu.VMEM((BATCH,), jnp.int32),
            pltpu.VMEM((BATCH,), jnp.int32),
            pltpu.VMEM((BATCH,), jnp.int32),
            pltpu.VMEM((BATCH,), jnp.int32),
            pltpu.VMEM((BATCH,), jnp.float32),
            pltpu.VMEM((BATCH, 128), jnp.float32),
            pltpu.VMEM((BATCH, 128), jnp.float32),
            pltpu.VMEM((AROWS * 128,), jnp.float32),
            pltpu.VMEM((AROWS + 8,), jnp.float32),
            pltpu.VMEM_SHARED((32 * MCAP,), jnp.int32),
            pltpu.VMEM_SHARED((32 * MCAP,), jnp.int32),
            pltpu.VMEM_SHARED((256,), jnp.int32),
            pltpu.SemaphoreType.DMA,
            pltpu.SemaphoreType.DMA,
        ],
    )(src, dst, asf, adf, h2d)


def kernel(x, edge_index, batch, W1, a_src1, a_dst1, b1, W2, a_src2, a_dst2,
           b2, fc1_W, fc1_b, fc2_W, fc2_b):
    src = edge_index[0]
    dst = edge_index[1]

    def _amap(a_src, a_dst):
        eye = jnp.eye(HH, dtype=jnp.float32)
        blk = jnp.repeat(eye, CH, axis=0)
        asrc_col = a_src.reshape(FF, 1) * blk
        adst_col = a_dst.reshape(FF, 1) * blk
        return jnp.concatenate([asrc_col, adst_col], axis=1)

    amap1 = _amap(a_src1, a_dst1)
    amap2 = _amap(a_src2, a_dst2)

    h1, as1, ad1, el1 = _tc_a1(x, W1, amap1)
    acc1, den1 = _sc_edge(src, dst, as1.reshape(NPAD * 4),
                          ad1.reshape(NPAD * 4), h1.reshape(NN * 8, 128))
    h2, as2, ad2, el2 = _tc_a2(acc1.reshape(8, NPAD, 128),
                               den1.reshape(8, NPAD).T, el1, h1,
                               b1.reshape(1, FF), W2, amap2)
    acc2, den2 = _sc_edge(src, dst, as2.reshape(NPAD * 4),
                          ad2.reshape(NPAD * 4), h2.reshape(NN * 8, 128))
    out = _tc_d(acc2.reshape(8, NPAD, 128), den2.reshape(8, NPAD).T, el2,
                h2,
                b2.reshape(1, FF), batch.reshape(NN // BR, 1, BR),
                fc1_W, fc1_b.reshape(1, CH), fc2_W, fc2_b.reshape(1, 10))
    return out
